# baseline (speedup 1.0000x reference)
import functools

import jax
import jax.numpy as jnp
from jax import lax
from jax.experimental import pallas as pl
from jax.experimental.pallas import tpu as pltpu
from jax.experimental.pallas import tpu_sc as plsc

F = 4096
Q = 4096
NC, NS, L = 2, 16, 16
NW = NC * NS
PPW = Q // NW
NG = PPW // L
NROW = 15
CHUNKS = (64, 64, 128, 256, 512, 1024, 2048)

_EPS = 1e-12


def _safe_div(num, den):
    return num / jnp.where(jnp.abs(den) > _EPS, den, 1.0)


def _closest_from_rows(rows, px, py, pz):
    ax, ay, az, bx, by, bz, cx, cy, cz, abx, aby, abz, acx, acy, acz = rows
    apx, apy, apz = px - ax, py - ay, pz - az
    d1 = abx * apx + aby * apy + abz * apz
    d2 = acx * apx + acy * apy + acz * apz
    bpx, bpy, bpz = px - bx, py - by, pz - bz
    d3 = abx * bpx + aby * bpy + abz * bpz
    d4 = acx * bpx + acy * bpy + acz * bpz
    cpx, cpy, cpz = px - cx, py - cy, pz - cz
    d5 = abx * cpx + aby * cpy + abz * cpz
    d6 = acx * cpx + acy * cpy + acz * cpz
    vc = d1 * d4 - d3 * d2
    vb = d5 * d2 - d1 * d6
    va = d3 * d6 - d5 * d4
    t_ab = _safe_div(d1, d1 - d3)
    t_ac = _safe_div(d2, d2 - d6)
    e_bc = d4 - d3
    f_bc = d5 - d6
    t_bc = _safe_div(e_bc, e_bc + f_bc)
    denom = va + vb + vc
    v_face = _safe_div(vb, denom)
    w_face = _safe_div(vc, denom)
    m1 = (d1 <= 0) & (d2 <= 0)
    m2 = (d3 >= 0) & (d4 <= d3)
    m3 = (vc <= 0) & (d1 >= 0) & (d3 <= 0)
    m4 = (d6 >= 0) & (d5 <= d6)
    m5 = (vb <= 0) & (d2 >= 0) & (d6 <= 0)
    m6 = (va <= 0) & (e_bc >= 0) & (f_bc >= 0)
    zero = jnp.zeros_like(d1)
    one = jnp.ones_like(d1)

    def _select(cases, default):
        out = default
        for m, val in reversed(cases):
            out = jnp.where(m, val, out)
        return out

    v = _select([(m1, zero), (m2, one), (m3, t_ab), (m4, zero),
                 (m5, zero), (m6, 1.0 - t_bc)], v_face)
    w = _select([(m1, zero), (m2, zero), (m3, zero), (m4, one),
                 (m5, t_ac), (m6, t_bc)], w_face)
    clx = ax + v * abx + w * acx
    cly = ay + v * aby + w * acy
    clz = az + v * abz + w * acz
    dx, dy, dz = px - clx, py - cly, pz - clz
    dist2 = dx * dx + dy * dy + dz * dz
    return dist2, clx, cly, clz


def _rsqrt(x):
    i = lax.bitcast_convert_type(x, jnp.int32)
    i = jnp.int32(0x5F3759DF) - lax.shift_right_arithmetic(i, 1)
    y = lax.bitcast_convert_type(i, jnp.float32)
    for _ in range(2):
        y = y * (1.5 - 0.5 * x * y * y)
    return y


def _sc_body(face_hbm, pts_hbm, out_d, out_c, out_f, *scratch):
    rows_v = scratch[:NROW]
    ux_v, uy_v, uz_v, ua_v, cand_v, pts_v, dist_v, clos_v, bidx_v = \
        scratch[NROW:]
    wid = lax.axis_index("s") * NC + lax.axis_index("c")
    base = wid * PPW
    for r in range(NROW):
        pltpu.sync_copy(face_hbm.at[pl.ds(r * F, F)], rows_v[r])
    pltpu.sync_copy(pts_hbm.at[:, pl.ds(base, PPW)], pts_v)

    def mk_normals(k, _):
        o = k * L
        csl = pl.ds(o, L)
        ax, ay, az = rows_v[0][csl], rows_v[1][csl], rows_v[2][csl]
        abx, aby, abz = rows_v[9][csl], rows_v[10][csl], rows_v[11][csl]
        acx, acy, acz = rows_v[12][csl], rows_v[13][csl], rows_v[14][csl]
        nx = aby * acz - abz * acy
        ny = abz * acx - abx * acz
        nz = abx * acy - aby * acx
        inv = _rsqrt(jnp.maximum(nx * nx + ny * ny + nz * nz, 1e-30))
        ux, uy, uz = nx * inv, ny * inv, nz * inv
        ux_v[csl] = ux
        uy_v[csl] = uy
        uz_v[csl] = uz
        ua_v[csl] = ux * ax + uy * ay + uz * az
        return 0

    lax.fori_loop(0, F // L, mk_normals, 0)

    lane0 = lax.iota(jnp.int32, L) == 0

    def group_body(g, _):
        sl = pl.ds(g * L, L)
        px = pts_v[0, sl]
        py = pts_v[1, sl]
        pz = pts_v[2, sl]

        def eval_face(ffull, carry):
            bd, bi = carry
            rows = [plsc.load_gather(rv, [ffull]) for rv in rows_v]
            dist2, _, _, _ = _closest_from_rows(rows, px, py, pz)
            m = dist2 < bd
            return jnp.where(m, dist2, bd), jnp.where(m, ffull, bi)

        def eval_direct(f, carry):
            return eval_face(jnp.full((L,), f, jnp.int32), carry)

        init = (jnp.full((L,), jnp.inf, jnp.float32),
                jnp.zeros((L,), jnp.int32))
        bd, bi = lax.fori_loop(0, CHUNKS[0], eval_direct, init)

        f0 = CHUNKS[0]
        for ch in CHUNKS[1:]:
            thr = bd * 1.001 + 1e-6

            @plsc.parallel_loop(f0, f0 + ch, 1,
                                carry=jnp.zeros((L,), jnp.int32))
            def test_face(f, cntv):
                ffull = jnp.full((L,), f, jnp.int32)
                ux = plsc.load_gather(ux_v, [ffull])
                uy = plsc.load_gather(uy_v, [ffull])
                uz = plsc.load_gather(uz_v, [ffull])
                ua = plsc.load_gather(ua_v, [ffull])
                dpl = ux * px + uy * py + uz * pz - ua
                interest = dpl * dpl < thr
                pc = plsc.all_reduce_population_count(interest)
                cand = pc > 0
                plsc.store_scatter(cand_v, [cntv], ffull,
                                   mask=lane0 & cand)
                return cntv + jnp.where(cand, 1, 0)

            cnt = jnp.max(test_face)

            def eval_cand(k, carry):
                cf = plsc.load_gather(cand_v, [jnp.full((L,), k, jnp.int32)])
                return eval_face(cf, carry)

            bd, bi = lax.fori_loop(0, cnt, eval_cand, (bd, bi))
            f0 += ch

        rows = [plsc.load_gather(rv, [bi]) for rv in rows_v]
        dist2, clx, cly, clz = _closest_from_rows(rows, px, py, pz)
        dist_v[sl] = dist2
        bidx_v[sl] = bi
        clos_v[0, sl] = clx
        clos_v[1, sl] = cly
        clos_v[2, sl] = clz
        return 0

    lax.fori_loop(0, NG, group_body, 0)

    pltpu.sync_copy(dist_v, out_d.at[pl.ds(base, PPW)])
    pltpu.sync_copy(bidx_v, out_f.at[pl.ds(base, PPW)])
    pltpu.sync_copy(clos_v, out_c.at[:, pl.ds(base, PPW)])


@functools.cache
def _sc_call():
    return functools.partial(
        pl.kernel,
        out_type=(
            jax.ShapeDtypeStruct((Q,), jnp.float32),
            jax.ShapeDtypeStruct((3, Q), jnp.float32),
            jax.ShapeDtypeStruct((Q,), jnp.int32),
        ),
        mesh=plsc.VectorSubcoreMesh(
            core_axis_name="c", subcore_axis_name="s",
            num_cores=NC, num_subcores=NS),
        scratch_types=(
            [pltpu.VMEM((F,), jnp.float32) for _ in range(NROW)] + [
                pltpu.VMEM((F,), jnp.float32),
                pltpu.VMEM((F,), jnp.float32),
                pltpu.VMEM((F,), jnp.float32),
                pltpu.VMEM((F,), jnp.float32),
                pltpu.VMEM((max(CHUNKS),), jnp.int32),
                pltpu.VMEM((3, PPW), jnp.float32),
                pltpu.VMEM((PPW,), jnp.float32),
                pltpu.VMEM((3, PPW), jnp.float32),
                pltpu.VMEM((PPW,), jnp.int32),
            ]),
        compiler_params=pltpu.CompilerParams(use_tc_tiling_on_sc=False,
                                             needs_layout_passes=False),
    )(_sc_body)


def kernel(triangles, points):
    tri = triangles[0]
    a = tri[:, 0, :]
    b = tri[:, 1, :]
    c = tri[:, 2, :]
    ab = b - a
    ac = c - a
    face = jnp.concatenate(
        [a.T, b.T, c.T, ab.T, ac.T], axis=0).reshape(-1)

    pts0 = points[0]
    qmin = pts0.min(axis=0)
    qrng = pts0.max(axis=0) - qmin + 1e-9
    qq = jnp.clip((pts0 - qmin) / qrng * 16.0, 0.0, 15.0).astype(jnp.int32)
    code = jnp.zeros((Q,), jnp.int32)
    for bit in range(4):
        for dim in range(3):
            code = code | (((qq[:, dim] >> bit) & 1) << (bit * 3 + dim))
    order = jnp.argsort(code)
    pts = pts0[order].T

    d, cl, fi = _sc_call()(face, pts)
    dq = jnp.zeros((Q,), jnp.float32).at[order].set(d)
    clq = jnp.zeros((Q, 3), jnp.float32).at[order].set(cl.T)
    fiq = jnp.zeros((Q,), jnp.int32).at[order].set(fi)
    return dq[None], clq[None], fiq[None]


# speedup vs baseline: 1.0726x; 1.0726x over previous
import functools

import jax
import jax.numpy as jnp
from jax import lax
from jax.experimental import pallas as pl
from jax.experimental.pallas import tpu as pltpu
from jax.experimental.pallas import tpu_sc as plsc

F = 4096
Q = 4096
NC, NS, L = 2, 16, 16
NW = NC * NS
PPW = Q // NW
NG = PPW // L
NROW = 15
CHUNKS = (64, 64, 128, 256, 512, 1024, 2048)

_EPS = 1e-12


def _safe_div(num, den):
    return num / jnp.where(jnp.abs(den) > _EPS, den, 1.0)


def _closest_from_rows(rows, px, py, pz):
    ax, ay, az, bx, by, bz, cx, cy, cz, abx, aby, abz, acx, acy, acz = rows
    apx, apy, apz = px - ax, py - ay, pz - az
    d1 = abx * apx + aby * apy + abz * apz
    d2 = acx * apx + acy * apy + acz * apz
    bpx, bpy, bpz = px - bx, py - by, pz - bz
    d3 = abx * bpx + aby * bpy + abz * bpz
    d4 = acx * bpx + acy * bpy + acz * bpz
    cpx, cpy, cpz = px - cx, py - cy, pz - cz
    d5 = abx * cpx + aby * cpy + abz * cpz
    d6 = acx * cpx + acy * cpy + acz * cpz
    vc = d1 * d4 - d3 * d2
    vb = d5 * d2 - d1 * d6
    va = d3 * d6 - d5 * d4
    t_ab = _safe_div(d1, d1 - d3)
    t_ac = _safe_div(d2, d2 - d6)
    e_bc = d4 - d3
    f_bc = d5 - d6
    t_bc = _safe_div(e_bc, e_bc + f_bc)
    denom = va + vb + vc
    v_face = _safe_div(vb, denom)
    w_face = _safe_div(vc, denom)
    m1 = (d1 <= 0) & (d2 <= 0)
    m2 = (d3 >= 0) & (d4 <= d3)
    m3 = (vc <= 0) & (d1 >= 0) & (d3 <= 0)
    m4 = (d6 >= 0) & (d5 <= d6)
    m5 = (vb <= 0) & (d2 >= 0) & (d6 <= 0)
    m6 = (va <= 0) & (e_bc >= 0) & (f_bc >= 0)
    zero = jnp.zeros_like(d1)
    one = jnp.ones_like(d1)

    def _select(cases, default):
        out = default
        for m, val in reversed(cases):
            out = jnp.where(m, val, out)
        return out

    v = _select([(m1, zero), (m2, one), (m3, t_ab), (m4, zero),
                 (m5, zero), (m6, 1.0 - t_bc)], v_face)
    w = _select([(m1, zero), (m2, zero), (m3, zero), (m4, one),
                 (m5, t_ac), (m6, t_bc)], w_face)
    clx = ax + v * abx + w * acx
    cly = ay + v * aby + w * acy
    clz = az + v * abz + w * acz
    dx, dy, dz = px - clx, py - cly, pz - clz
    dist2 = dx * dx + dy * dy + dz * dz
    return dist2, clx, cly, clz


def _rsqrt(x):
    i = lax.bitcast_convert_type(x, jnp.int32)
    i = jnp.int32(0x5F3759DF) - lax.shift_right_arithmetic(i, 1)
    y = lax.bitcast_convert_type(i, jnp.float32)
    for _ in range(2):
        y = y * (1.5 - 0.5 * x * y * y)
    return y


def _sc_body(face_hbm, pts_hbm, out_d, out_c, out_f, *scratch):
    rows_v = scratch[:NROW]
    ux_v, uy_v, uz_v, ua_v, cand_v, pts_v, dist_v, clos_v, bidx_v = \
        scratch[NROW:]
    wid = lax.axis_index("s") * NC + lax.axis_index("c")
    base = wid * PPW
    for r in range(NROW):
        pltpu.sync_copy(face_hbm.at[pl.ds(r * F, F)], rows_v[r])
    pltpu.sync_copy(pts_hbm.at[:, pl.ds(base, PPW)], pts_v)

    def mk_normals(k, _):
        o = k * L
        csl = pl.ds(o, L)
        ax, ay, az = rows_v[0][csl], rows_v[1][csl], rows_v[2][csl]
        abx, aby, abz = rows_v[9][csl], rows_v[10][csl], rows_v[11][csl]
        acx, acy, acz = rows_v[12][csl], rows_v[13][csl], rows_v[14][csl]
        nx = aby * acz - abz * acy
        ny = abz * acx - abx * acz
        nz = abx * acy - aby * acx
        inv = _rsqrt(jnp.maximum(nx * nx + ny * ny + nz * nz, 1e-30))
        ux, uy, uz = nx * inv, ny * inv, nz * inv
        ux_v[csl] = ux
        uy_v[csl] = uy
        uz_v[csl] = uz
        ua_v[csl] = ux * ax + uy * ay + uz * az
        return 0

    lax.fori_loop(0, F // L, mk_normals, 0)

    lane0 = lax.iota(jnp.int32, L) == 0

    def group_body(g, _):
        sl = pl.ds(g * L, L)
        px = pts_v[0, sl]
        py = pts_v[1, sl]
        pz = pts_v[2, sl]

        def eval_face(ffull, carry):
            bd, bi = carry
            rows = [plsc.load_gather(rv, [ffull]) for rv in rows_v]
            dist2, _, _, _ = _closest_from_rows(rows, px, py, pz)
            m = dist2 < bd
            return jnp.where(m, dist2, bd), jnp.where(m, ffull, bi)

        def eval_direct(f, carry):
            return eval_face(jnp.full((L,), f, jnp.int32), carry)

        init = (jnp.full((L,), jnp.inf, jnp.float32),
                jnp.zeros((L,), jnp.int32))
        bd, bi = plsc.parallel_loop(0, CHUNKS[0], 1, carry=init)(eval_direct)

        f0 = CHUNKS[0]
        for ch in CHUNKS[1:]:
            thr = bd * 1.001 + 1e-6

            @plsc.parallel_loop(f0, f0 + ch, 1,
                                carry=jnp.zeros((L,), jnp.int32))
            def test_face(f, cntv):
                ffull = jnp.full((L,), f, jnp.int32)
                ux = plsc.load_gather(ux_v, [ffull])
                uy = plsc.load_gather(uy_v, [ffull])
                uz = plsc.load_gather(uz_v, [ffull])
                ua = plsc.load_gather(ua_v, [ffull])
                dpl = ux * px + uy * py + uz * pz - ua
                interest = dpl * dpl < thr
                pc = plsc.all_reduce_population_count(interest)
                cand = pc > 0
                plsc.store_scatter(cand_v, [cntv], ffull,
                                   mask=lane0 & cand)
                return cntv + jnp.where(cand, 1, 0)

            cnt = jnp.max(test_face)

            def eval_cand(k, carry):
                cf = plsc.load_gather(cand_v, [jnp.full((L,), k, jnp.int32)])
                return eval_face(cf, carry)

            bd, bi = plsc.parallel_loop(0, cnt, 1,
                                        carry=(bd, bi))(eval_cand)
            f0 += ch

        rows = [plsc.load_gather(rv, [bi]) for rv in rows_v]
        dist2, clx, cly, clz = _closest_from_rows(rows, px, py, pz)
        dist_v[sl] = dist2
        bidx_v[sl] = bi
        clos_v[0, sl] = clx
        clos_v[1, sl] = cly
        clos_v[2, sl] = clz
        return 0

    lax.fori_loop(0, NG, group_body, 0)

    pltpu.sync_copy(dist_v, out_d.at[pl.ds(base, PPW)])
    pltpu.sync_copy(bidx_v, out_f.at[pl.ds(base, PPW)])
    pltpu.sync_copy(clos_v, out_c.at[:, pl.ds(base, PPW)])


@functools.cache
def _sc_call():
    return functools.partial(
        pl.kernel,
        out_type=(
            jax.ShapeDtypeStruct((Q,), jnp.float32),
            jax.ShapeDtypeStruct((3, Q), jnp.float32),
            jax.ShapeDtypeStruct((Q,), jnp.int32),
        ),
        mesh=plsc.VectorSubcoreMesh(
            core_axis_name="c", subcore_axis_name="s",
            num_cores=NC, num_subcores=NS),
        scratch_types=(
            [pltpu.VMEM((F,), jnp.float32) for _ in range(NROW)] + [
                pltpu.VMEM((F,), jnp.float32),
                pltpu.VMEM((F,), jnp.float32),
                pltpu.VMEM((F,), jnp.float32),
                pltpu.VMEM((F,), jnp.float32),
                pltpu.VMEM((max(CHUNKS),), jnp.int32),
                pltpu.VMEM((3, PPW), jnp.float32),
                pltpu.VMEM((PPW,), jnp.float32),
                pltpu.VMEM((3, PPW), jnp.float32),
                pltpu.VMEM((PPW,), jnp.int32),
            ]),
        compiler_params=pltpu.CompilerParams(use_tc_tiling_on_sc=False,
                                             needs_layout_passes=False),
    )(_sc_body)


def kernel(triangles, points):
    tri = triangles[0]
    a = tri[:, 0, :]
    b = tri[:, 1, :]
    c = tri[:, 2, :]
    ab = b - a
    ac = c - a
    face = jnp.concatenate(
        [a.T, b.T, c.T, ab.T, ac.T], axis=0).reshape(-1)

    pts = points[0].T
    d, cl, fi = _sc_call()(face, pts)
    return d[None], cl.T[None], fi[None]


# speedup vs baseline: 1.1426x; 1.0653x over previous
import functools

import jax
import jax.numpy as jnp
from jax import lax
from jax.experimental import pallas as pl
from jax.experimental.pallas import tpu as pltpu
from jax.experimental.pallas import tpu_sc as plsc

F = 4096
Q = 4096
NC, NS, L = 2, 16, 16
NW = NC * NS
PPW = Q // NW
NG = PPW // L
NROW = 15
CHUNKS = (64, 64, 128, 128, 256, 256, 512, 512, 512, 512, 512, 640)

_EPS = 1e-12


def _safe_div(num, den):
    return num / jnp.where(jnp.abs(den) > _EPS, den, 1.0)


def _closest_from_rows(rows, px, py, pz):
    ax, ay, az, bx, by, bz, cx, cy, cz, abx, aby, abz, acx, acy, acz = rows
    apx, apy, apz = px - ax, py - ay, pz - az
    d1 = abx * apx + aby * apy + abz * apz
    d2 = acx * apx + acy * apy + acz * apz
    bpx, bpy, bpz = px - bx, py - by, pz - bz
    d3 = abx * bpx + aby * bpy + abz * bpz
    d4 = acx * bpx + acy * bpy + acz * bpz
    cpx, cpy, cpz = px - cx, py - cy, pz - cz
    d5 = abx * cpx + aby * cpy + abz * cpz
    d6 = acx * cpx + acy * cpy + acz * cpz
    vc = d1 * d4 - d3 * d2
    vb = d5 * d2 - d1 * d6
    va = d3 * d6 - d5 * d4
    t_ab = _safe_div(d1, d1 - d3)
    t_ac = _safe_div(d2, d2 - d6)
    e_bc = d4 - d3
    f_bc = d5 - d6
    t_bc = _safe_div(e_bc, e_bc + f_bc)
    denom = va + vb + vc
    v_face = _safe_div(vb, denom)
    w_face = _safe_div(vc, denom)
    m1 = (d1 <= 0) & (d2 <= 0)
    m2 = (d3 >= 0) & (d4 <= d3)
    m3 = (vc <= 0) & (d1 >= 0) & (d3 <= 0)
    m4 = (d6 >= 0) & (d5 <= d6)
    m5 = (vb <= 0) & (d2 >= 0) & (d6 <= 0)
    m6 = (va <= 0) & (e_bc >= 0) & (f_bc >= 0)
    zero = jnp.zeros_like(d1)
    one = jnp.ones_like(d1)

    def _select(cases, default):
        out = default
        for m, val in reversed(cases):
            out = jnp.where(m, val, out)
        return out

    v = _select([(m1, zero), (m2, one), (m3, t_ab), (m4, zero),
                 (m5, zero), (m6, 1.0 - t_bc)], v_face)
    w = _select([(m1, zero), (m2, zero), (m3, zero), (m4, one),
                 (m5, t_ac), (m6, t_bc)], w_face)
    clx = ax + v * abx + w * acx
    cly = ay + v * aby + w * acy
    clz = az + v * abz + w * acz
    dx, dy, dz = px - clx, py - cly, pz - clz
    dist2 = dx * dx + dy * dy + dz * dz
    return dist2, clx, cly, clz


def _rsqrt(x):
    i = lax.bitcast_convert_type(x, jnp.int32)
    i = jnp.int32(0x5F3759DF) - lax.shift_right_arithmetic(i, 1)
    y = lax.bitcast_convert_type(i, jnp.float32)
    for _ in range(2):
        y = y * (1.5 - 0.5 * x * y * y)
    return y


def _sc_body(face_hbm, pts_hbm, out_d, out_c, out_f, *scratch):
    rows_v = scratch[:NROW]
    ux_v, uy_v, uz_v, ua_v, cand_v, pts_v, dist_v, clos_v, bidx_v, sem = \
        scratch[NROW:]
    wid = lax.axis_index("s") * NC + lax.axis_index("c")
    base = wid * PPW
    copies = [pltpu.async_copy(face_hbm.at[pl.ds(r * F, F)], rows_v[r], sem)
              for r in range(NROW)]
    copies.append(
        pltpu.async_copy(pts_hbm.at[:, pl.ds(base, PPW)], pts_v, sem))
    for cp in copies:
        cp.wait()

    def mk_normals(k, _):
        o = k * L
        csl = pl.ds(o, L)
        ax, ay, az = rows_v[0][csl], rows_v[1][csl], rows_v[2][csl]
        abx, aby, abz = rows_v[9][csl], rows_v[10][csl], rows_v[11][csl]
        acx, acy, acz = rows_v[12][csl], rows_v[13][csl], rows_v[14][csl]
        nx = aby * acz - abz * acy
        ny = abz * acx - abx * acz
        nz = abx * acy - aby * acx
        inv = _rsqrt(jnp.maximum(nx * nx + ny * ny + nz * nz, 1e-30))
        ux, uy, uz = nx * inv, ny * inv, nz * inv
        ux_v[csl] = ux
        uy_v[csl] = uy
        uz_v[csl] = uz
        ua_v[csl] = ux * ax + uy * ay + uz * az
        return 0

    lax.fori_loop(0, F // L, mk_normals, 0)

    lane0 = lax.iota(jnp.int32, L) == 0

    def group_body(g, _):
        sl = pl.ds(g * L, L)
        px = pts_v[0, sl]
        py = pts_v[1, sl]
        pz = pts_v[2, sl]

        def eval_face(ffull, carry):
            bd, bi = carry
            rows = [plsc.load_gather(rv, [ffull]) for rv in rows_v]
            dist2, _, _, _ = _closest_from_rows(rows, px, py, pz)
            m = dist2 < bd
            return jnp.where(m, dist2, bd), jnp.where(m, ffull, bi)

        def eval_direct(f, carry):
            return eval_face(jnp.full((L,), f, jnp.int32), carry)

        init = (jnp.full((L,), jnp.inf, jnp.float32),
                jnp.zeros((L,), jnp.int32))
        bd, bi = plsc.parallel_loop(0, CHUNKS[0], 1, carry=init)(eval_direct)

        f0 = CHUNKS[0]
        for ch in CHUNKS[1:]:
            thr = bd * 1.001 + 1e-6

            @plsc.parallel_loop(f0, f0 + ch, 1,
                                carry=jnp.zeros((L,), jnp.int32))
            def test_face(f, cntv):
                ffull = jnp.full((L,), f, jnp.int32)
                ux = plsc.load_gather(ux_v, [ffull])
                uy = plsc.load_gather(uy_v, [ffull])
                uz = plsc.load_gather(uz_v, [ffull])
                ua = plsc.load_gather(ua_v, [ffull])
                dpl = ux * px + uy * py + uz * pz - ua
                interest = dpl * dpl < thr
                pc = plsc.all_reduce_population_count(interest)
                cand = pc > 0
                plsc.store_scatter(cand_v, [cntv], ffull,
                                   mask=lane0 & cand)
                return cntv + jnp.where(cand, 1, 0)

            cnt = jnp.max(test_face)

            def eval_cand(k, carry):
                cf = plsc.load_gather(cand_v, [jnp.full((L,), k, jnp.int32)])
                return eval_face(cf, carry)

            bd, bi = plsc.parallel_loop(0, cnt, 1,
                                        carry=(bd, bi))(eval_cand)
            f0 += ch

        rows = [plsc.load_gather(rv, [bi]) for rv in rows_v]
        dist2, clx, cly, clz = _closest_from_rows(rows, px, py, pz)
        dist_v[sl] = dist2
        bidx_v[sl] = bi
        clos_v[0, sl] = clx
        clos_v[1, sl] = cly
        clos_v[2, sl] = clz
        return 0

    lax.fori_loop(0, NG, group_body, 0)

    pltpu.sync_copy(dist_v, out_d.at[pl.ds(base, PPW)])
    pltpu.sync_copy(bidx_v, out_f.at[pl.ds(base, PPW)])
    pltpu.sync_copy(clos_v, out_c.at[:, pl.ds(base, PPW)])


@functools.cache
def _sc_call():
    return functools.partial(
        pl.kernel,
        out_type=(
            jax.ShapeDtypeStruct((Q,), jnp.float32),
            jax.ShapeDtypeStruct((3, Q), jnp.float32),
            jax.ShapeDtypeStruct((Q,), jnp.int32),
        ),
        mesh=plsc.VectorSubcoreMesh(
            core_axis_name="c", subcore_axis_name="s",
            num_cores=NC, num_subcores=NS),
        scratch_types=(
            [pltpu.VMEM((F,), jnp.float32) for _ in range(NROW)] + [
                pltpu.VMEM((F,), jnp.float32),
                pltpu.VMEM((F,), jnp.float32),
                pltpu.VMEM((F,), jnp.float32),
                pltpu.VMEM((F,), jnp.float32),
                pltpu.VMEM((max(CHUNKS),), jnp.int32),
                pltpu.VMEM((3, PPW), jnp.float32),
                pltpu.VMEM((PPW,), jnp.float32),
                pltpu.VMEM((3, PPW), jnp.float32),
                pltpu.VMEM((PPW,), jnp.int32),
                pltpu.SemaphoreType.DMA,
            ]),
        compiler_params=pltpu.CompilerParams(use_tc_tiling_on_sc=False,
                                             needs_layout_passes=False),
    )(_sc_body)


def kernel(triangles, points):
    tri = triangles[0]
    a = tri[:, 0, :]
    b = tri[:, 1, :]
    c = tri[:, 2, :]
    ab = b - a
    ac = c - a
    face = jnp.concatenate(
        [a.T, b.T, c.T, ab.T, ac.T], axis=0).reshape(-1)

    pts = points[0].T
    d, cl, fi = _sc_call()(face, pts)
    return d[None], cl.T[None], fi[None]


# speedup vs baseline: 1.1428x; 1.0002x over previous
import functools

import jax
import jax.numpy as jnp
from jax import lax
from jax.experimental import pallas as pl
from jax.experimental.pallas import tpu as pltpu
from jax.experimental.pallas import tpu_sc as plsc

F = 4096
Q = 4096
NC, NS, L = 2, 16, 16
NW = NC * NS
PPW = Q // NW
NG = PPW // L
NROW = 15
CHUNKS = (64, 64, 128, 128, 256, 256, 512, 512, 512, 512, 512, 640)

_EPS = 1e-12


def _safe_div(num, den):
    return num / jnp.where(jnp.abs(den) > _EPS, den, 1.0)


def _closest_from_rows(rows, px, py, pz):
    ax, ay, az, bx, by, bz, cx, cy, cz, abx, aby, abz, acx, acy, acz = rows
    apx, apy, apz = px - ax, py - ay, pz - az
    d1 = abx * apx + aby * apy + abz * apz
    d2 = acx * apx + acy * apy + acz * apz
    bpx, bpy, bpz = px - bx, py - by, pz - bz
    d3 = abx * bpx + aby * bpy + abz * bpz
    d4 = acx * bpx + acy * bpy + acz * bpz
    cpx, cpy, cpz = px - cx, py - cy, pz - cz
    d5 = abx * cpx + aby * cpy + abz * cpz
    d6 = acx * cpx + acy * cpy + acz * cpz
    vc = d1 * d4 - d3 * d2
    vb = d5 * d2 - d1 * d6
    va = d3 * d6 - d5 * d4
    t_ab = _safe_div(d1, d1 - d3)
    t_ac = _safe_div(d2, d2 - d6)
    e_bc = d4 - d3
    f_bc = d5 - d6
    t_bc = _safe_div(e_bc, e_bc + f_bc)
    denom = va + vb + vc
    v_face = _safe_div(vb, denom)
    w_face = _safe_div(vc, denom)
    m1 = (d1 <= 0) & (d2 <= 0)
    m2 = (d3 >= 0) & (d4 <= d3)
    m3 = (vc <= 0) & (d1 >= 0) & (d3 <= 0)
    m4 = (d6 >= 0) & (d5 <= d6)
    m5 = (vb <= 0) & (d2 >= 0) & (d6 <= 0)
    m6 = (va <= 0) & (e_bc >= 0) & (f_bc >= 0)
    zero = jnp.zeros_like(d1)
    one = jnp.ones_like(d1)

    def _select(cases, default):
        out = default
        for m, val in reversed(cases):
            out = jnp.where(m, val, out)
        return out

    v = _select([(m1, zero), (m2, one), (m3, t_ab), (m4, zero),
                 (m5, zero), (m6, 1.0 - t_bc)], v_face)
    w = _select([(m1, zero), (m2, zero), (m3, zero), (m4, one),
                 (m5, t_ac), (m6, t_bc)], w_face)
    clx = ax + v * abx + w * acx
    cly = ay + v * aby + w * acy
    clz = az + v * abz + w * acz
    dx, dy, dz = px - clx, py - cly, pz - clz
    dist2 = dx * dx + dy * dy + dz * dz
    return dist2, clx, cly, clz


def _rsqrt(x):
    i = lax.bitcast_convert_type(x, jnp.int32)
    i = jnp.int32(0x5F3759DF) - lax.shift_right_arithmetic(i, 1)
    y = lax.bitcast_convert_type(i, jnp.float32)
    for _ in range(2):
        y = y * (1.5 - 0.5 * x * y * y)
    return y


def _sc_body(face_hbm, pts_hbm, out_d, out_c, out_f, *scratch):
    rows_v = scratch[:NROW]
    ux_v, uy_v, uz_v, ua_v, cand_v, pts_v, dist_v, clos_v, bidx_v, sem = \
        scratch[NROW:]
    wid = lax.axis_index("s") * NC + lax.axis_index("c")
    base = wid * PPW
    copies = [pltpu.async_copy(face_hbm.at[pl.ds(r * F, F)], rows_v[r], sem)
              for r in range(NROW)]
    copies.append(
        pltpu.async_copy(pts_hbm.at[:, pl.ds(base, PPW)], pts_v, sem))
    for cp in copies:
        cp.wait()

    def mk_normals(k, _):
        o = k * L
        csl = pl.ds(o, L)
        ax, ay, az = rows_v[0][csl], rows_v[1][csl], rows_v[2][csl]
        abx, aby, abz = rows_v[9][csl], rows_v[10][csl], rows_v[11][csl]
        acx, acy, acz = rows_v[12][csl], rows_v[13][csl], rows_v[14][csl]
        nx = aby * acz - abz * acy
        ny = abz * acx - abx * acz
        nz = abx * acy - aby * acx
        n2 = nx * nx + ny * ny + nz * nz
        inv = jnp.where(n2 > 1e-16, _rsqrt(jnp.maximum(n2, 1e-30)), 0.0)
        ux, uy, uz = nx * inv, ny * inv, nz * inv
        ux_v[csl] = ux
        uy_v[csl] = uy
        uz_v[csl] = uz
        ua_v[csl] = ux * ax + uy * ay + uz * az
        return 0

    lax.fori_loop(0, F // L, mk_normals, 0)

    lane0 = lax.iota(jnp.int32, L) == 0

    def group_body(g, _):
        sl = pl.ds(g * L, L)
        px = pts_v[0, sl]
        py = pts_v[1, sl]
        pz = pts_v[2, sl]

        def eval_face(ffull, carry):
            bd, bi = carry
            rows = [plsc.load_gather(rv, [ffull]) for rv in rows_v]
            dist2, _, _, _ = _closest_from_rows(rows, px, py, pz)
            m = dist2 < bd
            return jnp.where(m, dist2, bd), jnp.where(m, ffull, bi)

        def eval_direct(f, carry):
            return eval_face(jnp.full((L,), f, jnp.int32), carry)

        init = (jnp.full((L,), jnp.inf, jnp.float32),
                jnp.zeros((L,), jnp.int32))
        bd, bi = plsc.parallel_loop(0, CHUNKS[0], 1, carry=init)(eval_direct)

        f0 = CHUNKS[0]
        for ch in CHUNKS[1:]:
            thr = bd * 1.001 + 1e-6

            @plsc.parallel_loop(f0, f0 + ch, 1,
                                carry=jnp.zeros((L,), jnp.int32))
            def test_face(f, cntv):
                ffull = jnp.full((L,), f, jnp.int32)
                ux = plsc.load_gather(ux_v, [ffull])
                uy = plsc.load_gather(uy_v, [ffull])
                uz = plsc.load_gather(uz_v, [ffull])
                ua = plsc.load_gather(ua_v, [ffull])
                dpl = ux * px + uy * py + uz * pz - ua
                interest = dpl * dpl < thr
                pc = plsc.all_reduce_population_count(interest)
                cand = pc > 0
                plsc.store_scatter(cand_v, [cntv], ffull,
                                   mask=lane0 & cand)
                return cntv + jnp.where(cand, 1, 0)

            cnt = jnp.max(test_face)

            def eval_cand(k, carry):
                cf = plsc.load_gather(cand_v, [jnp.full((L,), k, jnp.int32)])
                return eval_face(cf, carry)

            bd, bi = plsc.parallel_loop(0, cnt, 1,
                                        carry=(bd, bi))(eval_cand)
            f0 += ch

        rows = [plsc.load_gather(rv, [bi]) for rv in rows_v]
        dist2, clx, cly, clz = _closest_from_rows(rows, px, py, pz)
        dist_v[sl] = dist2
        bidx_v[sl] = bi
        clos_v[0, sl] = clx
        clos_v[1, sl] = cly
        clos_v[2, sl] = clz
        return 0

    lax.fori_loop(0, NG, group_body, 0)

    pltpu.sync_copy(dist_v, out_d.at[pl.ds(base, PPW)])
    pltpu.sync_copy(bidx_v, out_f.at[pl.ds(base, PPW)])
    pltpu.sync_copy(clos_v, out_c.at[:, pl.ds(base, PPW)])


@functools.cache
def _sc_call():
    return functools.partial(
        pl.kernel,
        out_type=(
            jax.ShapeDtypeStruct((Q,), jnp.float32),
            jax.ShapeDtypeStruct((3, Q), jnp.float32),
            jax.ShapeDtypeStruct((Q,), jnp.int32),
        ),
        mesh=plsc.VectorSubcoreMesh(
            core_axis_name="c", subcore_axis_name="s",
            num_cores=NC, num_subcores=NS),
        scratch_types=(
            [pltpu.VMEM((F,), jnp.float32) for _ in range(NROW)] + [
                pltpu.VMEM((F,), jnp.float32),
                pltpu.VMEM((F,), jnp.float32),
                pltpu.VMEM((F,), jnp.float32),
                pltpu.VMEM((F,), jnp.float32),
                pltpu.VMEM((max(CHUNKS),), jnp.int32),
                pltpu.VMEM((3, PPW), jnp.float32),
                pltpu.VMEM((PPW,), jnp.float32),
                pltpu.VMEM((3, PPW), jnp.float32),
                pltpu.VMEM((PPW,), jnp.int32),
                pltpu.SemaphoreType.DMA,
            ]),
        compiler_params=pltpu.CompilerParams(use_tc_tiling_on_sc=False,
                                             needs_layout_passes=False),
    )(_sc_body)


def kernel(triangles, points):
    tri = triangles[0]
    a = tri[:, 0, :]
    b = tri[:, 1, :]
    c = tri[:, 2, :]
    ab = b - a
    ac = c - a
    face = jnp.concatenate(
        [a.T, b.T, c.T, ab.T, ac.T], axis=0).reshape(-1)

    pts = points[0].T
    d, cl, fi = _sc_call()(face, pts)
    return d[None], cl.T[None], fi[None]
